# baseline (speedup 1.0000x reference)
import jax
import jax.numpy as jnp
import numpy as np
from jax.experimental import pallas as pl

_KS = 5
_SIGMA = 2.0
_THRESH = 0.3
_H = 224
_W = 398
_BB = 2


def _gauss2d_bf16():
    ax = np.arange(_KS, dtype=np.float32) - np.float32((_KS - 1) / 2.0)
    g = np.exp(-(ax ** 2) / np.float32(2.0 * _SIGMA ** 2)).astype(np.float32)
    g = g / g.sum(dtype=np.float32)
    g2 = np.outer(g, g).astype(np.float32)
    import ml_dtypes
    return g2.astype(ml_dtypes.bfloat16).astype(np.float32)


def _band_matrices():
    import ml_dtypes
    g2b = _gauss2d_bf16()
    ms = np.zeros((3, _H, _H + 4), dtype=np.float32)
    for j in range(3):
        for i in range(_KS):
            for r in range(_H):
                ms[j, r, r + i] = g2b[i, j]
    return ms.astype(ml_dtypes.bfloat16)


def _nms_block_kernel(x_ref, m_ref, o_ref):
    xb = x_ref[...]
    xb = jnp.concatenate(
        [xb[..., 2:3], xb[..., 1:2], xb, xb[..., -2:-1], xb[..., -3:-2]],
        axis=3)
    xb = jnp.concatenate(
        [xb[:, :, 2:3], xb[:, :, 1:2], xb, xb[:, :, -2:-1], xb[:, :, -3:-2]],
        axis=2)
    m0 = m_ref[0]
    m1 = m_ref[1]
    m2 = m_ref[2]
    dn = (((1,), (0,)), ((), ()))
    ninf = jnp.float32(-jnp.inf)
    rpad = jnp.full((1, _W), ninf, dtype=jnp.float32)
    cpad = jnp.full((_H, 1), ninf, dtype=jnp.float32)
    for b in range(_BB):
      for c in range(2):
        xp = xb[b, c]
        y0 = jax.lax.dot_general(m0, xp, dn,
                                 preferred_element_type=jnp.float32)
        y1 = jax.lax.dot_general(m1, xp, dn,
                                 preferred_element_type=jnp.float32)
        y2 = jax.lax.dot_general(m2, xp, dn,
                                 preferred_element_type=jnp.float32)
        blurred = (y0[:, 0:_W] + y1[:, 1:_W + 1] + y2[:, 2:_W + 2]
                   + y1[:, 3:_W + 3] + y0[:, 4:_W + 4])
        bp = jnp.concatenate([rpad, blurred, rpad], axis=0)
        mv = jnp.maximum(jnp.maximum(bp[0:_H], bp[1:_H + 1]), bp[2:_H + 2])
        mp = jnp.concatenate([cpad, mv, cpad], axis=1)
        maxes = jnp.maximum(jnp.maximum(mp[:, 0:_W], mp[:, 1:_W + 1]),
                            mp[:, 2:_W + 2])
        keep = (blurred == maxes) & (blurred > _THRESH)
        o_ref[b, c] = jnp.where(keep, blurred, jnp.float32(0.0))


def kernel(hands_batch):
    B, C, H, W = hands_batch.shape
    xb = hands_batch.astype(jnp.bfloat16)
    m = jnp.asarray(_band_matrices())
    grid = (B // _BB,)
    spec = pl.BlockSpec((_BB, C, H, W), lambda i: (i, 0, 0, 0))
    mspec = pl.BlockSpec((3, _H, _H + 4), lambda i: (0, 0, 0))
    out = pl.pallas_call(
        _nms_block_kernel,
        grid=grid,
        in_specs=[spec, mspec],
        out_specs=spec,
        out_shape=jax.ShapeDtypeStruct((B, C, H, W), jnp.float32),
    )(xb, m)
    return out


# speedup vs baseline: 1.1344x; 1.1344x over previous
import jax
import jax.numpy as jnp
import numpy as np
from jax.experimental import pallas as pl
from jax.experimental.pallas import tpu as pltpu

_KS = 5
_SIGMA = 2.0
_THRESH = 0.3
_H = 224
_W = 398
_BLK = 4


def _gauss2d_bf16():
    ax = np.arange(_KS, dtype=np.float32) - np.float32((_KS - 1) / 2.0)
    g = np.exp(-(ax ** 2) / np.float32(2.0 * _SIGMA ** 2)).astype(np.float32)
    g = g / g.sum(dtype=np.float32)
    g2 = np.outer(g, g).astype(np.float32)
    import ml_dtypes
    return g2.astype(ml_dtypes.bfloat16).astype(np.float32)


def _band_matrices():
    import ml_dtypes
    g2b = _gauss2d_bf16()
    ms = np.zeros((3, _H, _H + 4), dtype=np.float32)
    for j in range(3):
        for i in range(_KS):
            for r in range(_H):
                ms[j, r, r + i] = g2b[i, j]
    return ms.astype(ml_dtypes.bfloat16)


def _nms_block_kernel(x_ref, m_ref, o_ref):
    x = x_ref[...]
    xb = x.astype(jnp.bfloat16)
    xb = jnp.concatenate(
        [xb[:, :, 2:3], xb[:, :, 1:2], xb, xb[:, :, -2:-1], xb[:, :, -3:-2]],
        axis=2)
    xb = jnp.concatenate(
        [xb[:, 2:3], xb[:, 1:2], xb, xb[:, -2:-1], xb[:, -3:-2]], axis=1)
    m0 = m_ref[0]
    m1 = m_ref[1]
    m2 = m_ref[2]
    dn = (((1,), (0,)), ((), ()))
    ninf = jnp.float32(-jnp.inf)
    rpad = jnp.full((1, _W), ninf, dtype=jnp.float32)
    cpad = jnp.full((_H, 1), ninf, dtype=jnp.float32)
    for b in range(_BLK):
        xp = xb[b]
        y0 = jax.lax.dot_general(m0, xp, dn,
                                 preferred_element_type=jnp.float32)
        y1 = jax.lax.dot_general(m1, xp, dn,
                                 preferred_element_type=jnp.float32)
        y2 = jax.lax.dot_general(m2, xp, dn,
                                 preferred_element_type=jnp.float32)
        blurred = (y0[:, 0:_W] + y1[:, 1:_W + 1] + y2[:, 2:_W + 2]
                   + y1[:, 3:_W + 3] + y0[:, 4:_W + 4])
        bp = jnp.concatenate([rpad, blurred, rpad], axis=0)
        mv = jnp.maximum(jnp.maximum(bp[0:_H], bp[1:_H + 1]), bp[2:_H + 2])
        mp = jnp.concatenate([cpad, mv, cpad], axis=1)
        maxes = jnp.maximum(jnp.maximum(mp[:, 0:_W], mp[:, 1:_W + 1]),
                            mp[:, 2:_W + 2])
        keep = (blurred == maxes) & (blurred > _THRESH)
        o_ref[b] = jnp.where(keep, blurred, jnp.float32(0.0))


def kernel(hands_batch):
    B, C, H, W = hands_batch.shape
    n = B * C
    x = hands_batch.reshape(n, H, W)
    m = jnp.asarray(_band_matrices())
    grid = (n // _BLK,)
    spec = pl.BlockSpec((_BLK, H, W), lambda i: (i, 0, 0))
    mspec = pl.BlockSpec((3, _H, _H + 4), lambda i: (0, 0, 0))
    out = pl.pallas_call(
        _nms_block_kernel,
        grid=grid,
        in_specs=[spec, mspec],
        out_specs=spec,
        out_shape=jax.ShapeDtypeStruct((n, H, W), jnp.float32),
        compiler_params=pltpu.CompilerParams(
            dimension_semantics=("parallel",)),
    )(x, m)
    return out.reshape(B, C, H, W)
